# baseline (speedup 1.0000x reference)
import functools

import jax
import jax.numpy as jnp
from jax import lax
from jax.experimental import pallas as pl
from jax.experimental.pallas import tpu as pltpu
from jax.experimental.pallas import tpu_sc as plsc

N = 50000
E = 800000
DIN = 640
H = 128
G = 64
NC, NS, L = 2, 16, 16
EPAD = 819200
ROWS = EPAD // 128
PAD = EPAD - E
NCHUNK = 4
CW = H // NCHUNK
ACC_ROWS = N + 16
RB = 1000
NBLK = N // RB
DEGN = 51200


def _mesh():
    return plsc.VectorSubcoreMesh(
        core_axis_name="c", subcore_axis_name="s", num_cores=NC,
        num_subcores=NS)


def _deg(dstr):
    @functools.partial(
        pl.kernel,
        out_type=jax.ShapeDtypeStruct((NC, DEGN), jnp.float32),
        mesh=_mesh(),
        scratch_types=[
            pltpu.VMEM_SHARED((DEGN,), jnp.float32),
            pltpu.VMEM((3200,), jnp.float32),
            pltpu.VMEM((128,), jnp.float32),
            pltpu.VMEM((8, 128), jnp.int32),
        ],
        compiler_params=pltpu.CompilerParams(use_tc_tiling_on_sc=False),
    )
    def deg_kernel(dst_hbm, degp_hbm, degacc, zb, ob, db):
        c = lax.axis_index("c")
        s = lax.axis_index("s")

        def fill(j, _):
            zb[pl.ds(j * L, L)] = jnp.zeros((L,), jnp.float32)
            return 0
        lax.fori_loop(0, 3200 // L, fill, 0)

        def fill1(j, _):
            ob[pl.ds(j * L, L)] = jnp.ones((L,), jnp.float32)
            return 0
        lax.fori_loop(0, 128 // L, fill1, 0)

        pltpu.sync_copy(zb, degacc.at[pl.ds(s * 3200, 3200)])
        plsc.subcore_barrier()

        base = c * (ROWS // NC) + s * 200

        def grp(g, _):
            r0 = base + g * 8
            pltpu.sync_copy(dst_hbm.at[pl.ds(r0, 8), :], db)
            for j in range(8):
                pltpu.sync_copy(ob, degacc.at[db.at[j]], add=True)
            return 0
        lax.fori_loop(0, 25, grp, 0)

        plsc.subcore_barrier()
        pltpu.sync_copy(degacc.at[pl.ds(s * 3200, 3200)],
                        degp_hbm.at[c, pl.ds(s * 3200, 3200)])

    return deg_kernel(dstr)


GR = 2
GE = GR * 128
NG = (ROWS // NS) // GR


def _agg(ypflat, src4f, dstr, zeros):
    @functools.partial(
        pl.kernel,
        out_type=jax.ShapeDtypeStruct((N, NCHUNK, CW), jnp.float32),
        mesh=_mesh(),
        scratch_types=[
            pltpu.VMEM_SHARED((ACC_ROWS, CW), jnp.float32),
            pltpu.VMEM((4 * GE,), jnp.int32),
            pltpu.VMEM((4, GR, 128), jnp.int32),
            pltpu.VMEM((2 * GE, CW), jnp.float32),
            pltpu.SemaphoreType.DMA,
            pltpu.SemaphoreType.DMA,
            pltpu.SemaphoreType.DMA,
        ],
        compiler_params=pltpu.CompilerParams(use_tc_tiling_on_sc=False),
    )
    def agg_kernel(yp_hbm, src_hbm, dst_hbm, zero_hbm, out_hbm,
                   acc, sb, db, gb, sem_i, sem_g, sem_s):
        c = lax.axis_index("c")
        s = lax.axis_index("s")

        def idx_load(chunk, g, slot):
            e0 = (s * (ROWS // NS) + g * GR) * 128
            pltpu.async_copy(src_hbm.at[chunk, pl.ds(e0, GE)],
                             sb.at[pl.ds(slot * GE, GE)], sem_i)
            pltpu.async_copy(
                dst_hbm.at[pl.ds(s * (ROWS // NS) + g * GR, GR), :],
                db.at[slot], sem_i)

        def idx_wait():
            pltpu.make_async_copy(src_hbm.at[0, pl.ds(0, GE)],
                                  sb.at[pl.ds(0, GE)], sem_i).wait()
            pltpu.make_async_copy(dst_hbm.at[pl.ds(0, GR), :],
                                  db.at[0], sem_i).wait()

        def gather_issue(slot4, slot2):
            pltpu.async_copy(yp_hbm.at[sb.at[pl.ds(slot4 * GE, GE)]],
                             gb.at[pl.ds(slot2 * GE, GE), :], sem_g)

        def gather_wait():
            pltpu.make_async_copy(yp_hbm.at[sb.at[pl.ds(0, GE)]],
                                  gb.at[pl.ds(0, GE), :], sem_g).wait()

        def scatter_issue(slot4, slot2):
            for j in range(GR):
                pltpu.async_copy(gb.at[pl.ds(slot2 * GE + j * 128, 128), :],
                                 acc.at[db.at[slot4, j]], sem_s, add=True)

        def scatter_wait():
            for j in range(GR):
                pltpu.make_async_copy(gb.at[pl.ds(j * 128, 128), :],
                                      acc.at[db.at[0, 0]], sem_s).wait()

        def per_chunk(k, _):
            chunk = c * 2 + k
            @pl.when(s < NS - 1)
            def _():
                pltpu.sync_copy(zero_hbm,
                                acc.at[pl.ds(s * 3200, 3200), :])
            @pl.when(s == NS - 1)
            def _():
                pltpu.sync_copy(zero_hbm.at[pl.ds(0, 2016), :],
                                acc.at[pl.ds(48000, 2016), :])
            plsc.subcore_barrier()

            idx_load(chunk, 0, 0)
            idx_load(chunk, 1, 1)

            def grp(g, _):
                slot2 = lax.rem(g, 2)
                slot4 = lax.rem(g, 4)

                @pl.when(g >= 2)
                def _():
                    scatter_wait()

                @pl.when(g < NG - 2)
                def _():
                    idx_load(chunk, g + 2, lax.rem(g + 2, 4))

                idx_wait()
                gather_issue(slot4, slot2)

                @pl.when(g >= 1)
                def _():
                    gather_wait()
                    scatter_issue(lax.rem(g - 1, 4), lax.rem(g - 1, 2))
                return 0
            lax.fori_loop(0, NG, grp, 0, unroll=False)

            gather_wait()
            scatter_issue(lax.rem(NG - 1, 4), lax.rem(NG - 1, 2))
            scatter_wait()
            scatter_wait()

            plsc.subcore_barrier()
            @pl.when(s < NS - 1)
            def _():
                pltpu.sync_copy(acc.at[pl.ds(s * 3200, 3200), :],
                                out_hbm.at[pl.ds(s * 3200, 3200), chunk, :])
            @pl.when(s == NS - 1)
            def _():
                pltpu.sync_copy(acc.at[pl.ds(48000, 2000), :],
                                out_hbm.at[pl.ds(48000, 2000), chunk, :])
            plsc.subcore_barrier()
            return 0
        lax.fori_loop(0, 2, per_chunk, 0)

    return agg_kernel(ypflat, src4f, dstr, zeros)


def _dis_of(degp_ref):
    deg = degp_ref[:, 0] + degp_ref[:, 1] + 1.0
    return lax.rsqrt(deg)


def _mm1(emb, degp, W1):
    def body(emb_ref, degp_ref, w_ref, o_ref):
        dis = _dis_of(degp_ref)
        acc = jnp.dot(emb_ref[...], w_ref[...],
                      preferred_element_type=jnp.float32)
        o_ref[...] = acc * dis[:, None]

    return pl.pallas_call(
        body,
        grid=(NBLK,),
        in_specs=[
            pl.BlockSpec((RB, DIN), lambda i: (i, 0)),
            pl.BlockSpec((RB, NC), lambda i: (i, 0)),
            pl.BlockSpec((DIN, H), lambda i: (0, 0)),
        ],
        out_specs=pl.BlockSpec((RB, H), lambda i: (i, 0)),
        out_shape=jax.ShapeDtypeStruct((N, H), jnp.float32),
    )(emb, degp, W1)


def _mm2(S1, yp, degp, W3, b1r):
    def body(s_ref, yp_ref, degp_ref, w_ref, b_ref, o_ref):
        dis = _dis_of(degp_ref)
        h = jnp.maximum((s_ref[...] + yp_ref[...]) * dis[:, None] + b_ref[...],
                        0.0)
        acc = jnp.dot(h, w_ref[...], preferred_element_type=jnp.float32)
        o_ref[...] = acc * dis[:, None]

    return pl.pallas_call(
        body,
        grid=(NBLK,),
        in_specs=[
            pl.BlockSpec((RB, H), lambda i: (i, 0)),
            pl.BlockSpec((RB, H), lambda i: (i, 0)),
            pl.BlockSpec((RB, NC), lambda i: (i, 0)),
            pl.BlockSpec((H, H), lambda i: (0, 0)),
            pl.BlockSpec((1, H), lambda i: (0, 0)),
        ],
        out_specs=pl.BlockSpec((RB, H), lambda i: (i, 0)),
        out_shape=jax.ShapeDtypeStruct((N, H), jnp.float32),
    )(S1, yp, degp, W3, b1r)


def _pool(S2, zp, degp, b3r, batch2d):
    def body(s_ref, zp_ref, degp_ref, b_ref, bat_ref, o_ref, acc, cnt):
        i = pl.program_id(0)

        @pl.when(i == 0)
        def _():
            acc[...] = jnp.zeros_like(acc)
            cnt[...] = jnp.zeros_like(cnt)

        dis = _dis_of(degp_ref)
        x2 = (s_ref[...] + zp_ref[...]) * dis[:, None] + b_ref[...]
        gi = lax.broadcasted_iota(jnp.int32, (RB, G), 1)
        oh_t = (bat_ref[...] == gi).astype(jnp.float32)
        dn = (((0,), (0,)), ((), ()))
        acc[...] += lax.dot_general(oh_t, x2, dn,
                                    preferred_element_type=jnp.float32)
        cnt[...] += lax.dot_general(oh_t, jnp.ones_like(x2), dn,
                                    preferred_element_type=jnp.float32)

        @pl.when(i == NBLK - 1)
        def _():
            o_ref[...] = acc[...] / jnp.maximum(cnt[...], 1.0)

    return pl.pallas_call(
        body,
        grid=(NBLK,),
        in_specs=[
            pl.BlockSpec((RB, H), lambda i: (i, 0)),
            pl.BlockSpec((RB, H), lambda i: (i, 0)),
            pl.BlockSpec((RB, NC), lambda i: (i, 0)),
            pl.BlockSpec((1, H), lambda i: (0, 0)),
            pl.BlockSpec((RB, 1), lambda i: (i, 0)),
        ],
        out_specs=pl.BlockSpec((G, H), lambda i: (0, 0)),
        out_shape=jax.ShapeDtypeStruct((G, H), jnp.float32),
        scratch_shapes=[
            pltpu.VMEM((G, H), jnp.float32),
            pltpu.VMEM((G, H), jnp.float32),
        ],
    )(S2, zp, degp, b3r, batch2d)


def kernel(emb, edge_index, batch, W1, b1, W3, b3):
    src = edge_index[0].astype(jnp.int32)
    dst = edge_index[1].astype(jnp.int32)
    ar = jnp.arange(PAD, dtype=jnp.int32)
    srcp = jnp.concatenate([src, (ar * 13) % N])
    dstp = jnp.concatenate([dst, N + (ar % 16)])
    src4 = (srcp[None, :] * NCHUNK
            + jnp.arange(NCHUNK, dtype=jnp.int32)[:, None]
            ).reshape(NCHUNK, ROWS, 128)
    dstr = dstp.reshape(ROWS, 128)

    zeros = jnp.zeros((3200, CW), jnp.float32)
    degp = _deg(dstr).T
    yp = _mm1(emb, degp, W1)
    src4f = src4.reshape(NCHUNK, EPAD)
    S1 = _agg(yp.reshape(NCHUNK * N, CW), src4f, dstr, zeros).reshape(N, H)
    zp = _mm2(S1, yp, degp, W3, b1.reshape(1, H))
    S2 = _agg(zp.reshape(NCHUNK * N, CW), src4f, dstr, zeros).reshape(N, H)
    return _pool(S2, zp, degp, b3.reshape(1, H),
                 batch.astype(jnp.int32).reshape(N, 1))


# speedup vs baseline: 1.1253x; 1.1253x over previous
import functools

import jax
import jax.numpy as jnp
from jax import lax
from jax.experimental import pallas as pl
from jax.experimental.pallas import tpu as pltpu
from jax.experimental.pallas import tpu_sc as plsc

N = 50000
E = 800000
DIN = 640
H = 128
G = 64
NC, NS, L = 2, 16, 16
EPAD = 819200
ROWS = EPAD // 128
PAD = EPAD - E
NCHUNK = 4
CW = H // NCHUNK
ACC_ROWS = N + 16
RB = 1000
NBLK = N // RB
DEGN = 51200


def _mesh():
    return plsc.VectorSubcoreMesh(
        core_axis_name="c", subcore_axis_name="s", num_cores=NC,
        num_subcores=NS)


def _deg(dstr):
    @functools.partial(
        pl.kernel,
        out_type=jax.ShapeDtypeStruct((NC, DEGN), jnp.float32),
        mesh=_mesh(),
        scratch_types=[
            pltpu.VMEM_SHARED((DEGN,), jnp.float32),
            pltpu.VMEM((3200,), jnp.float32),
            pltpu.VMEM((128,), jnp.float32),
            pltpu.VMEM((8, 128), jnp.int32),
        ],
        compiler_params=pltpu.CompilerParams(use_tc_tiling_on_sc=False),
    )
    def deg_kernel(dst_hbm, degp_hbm, degacc, zb, ob, db):
        c = lax.axis_index("c")
        s = lax.axis_index("s")

        def fill(j, _):
            zb[pl.ds(j * L, L)] = jnp.zeros((L,), jnp.float32)
            return 0
        lax.fori_loop(0, 3200 // L, fill, 0)

        def fill1(j, _):
            ob[pl.ds(j * L, L)] = jnp.ones((L,), jnp.float32)
            return 0
        lax.fori_loop(0, 128 // L, fill1, 0)

        pltpu.sync_copy(zb, degacc.at[pl.ds(s * 3200, 3200)])
        plsc.subcore_barrier()

        base = c * (ROWS // NC) + s * 200

        def grp(g, _):
            r0 = base + g * 8
            pltpu.sync_copy(dst_hbm.at[pl.ds(r0, 8), :], db)
            for j in range(8):
                pltpu.sync_copy(ob, degacc.at[db.at[j]], add=True)
            return 0
        lax.fori_loop(0, 25, grp, 0)

        plsc.subcore_barrier()
        pltpu.sync_copy(degacc.at[pl.ds(s * 3200, 3200)],
                        degp_hbm.at[c, pl.ds(s * 3200, 3200)])

    return deg_kernel(dstr)


GR = 2
GE = GR * 128
NG = (ROWS // NS) // GR


def _agg(ypflat, src4f, dstr, zeros):
    @functools.partial(
        pl.kernel,
        out_type=jax.ShapeDtypeStruct((N, NCHUNK, CW), jnp.float32),
        mesh=_mesh(),
        scratch_types=[
            pltpu.VMEM_SHARED((ACC_ROWS, CW), jnp.float32),
            pltpu.VMEM((6 * GE,), jnp.int32),
            pltpu.VMEM((6, GR, 128), jnp.int32),
            pltpu.VMEM((3 * GE, CW), jnp.float32),
            pltpu.SemaphoreType.DMA,
            pltpu.SemaphoreType.DMA,
            pltpu.SemaphoreType.DMA,
        ],
        compiler_params=pltpu.CompilerParams(use_tc_tiling_on_sc=False),
    )
    def agg_kernel(yp_hbm, src_hbm, dst_hbm, zero_hbm, out_hbm,
                   acc, sb, db, gb, sem_i, sem_g, sem_s):
        c = lax.axis_index("c")
        s = lax.axis_index("s")

        def idx_load(chunk, g, slot):
            e0 = (s * (ROWS // NS) + g * GR) * 128
            pltpu.async_copy(src_hbm.at[chunk, pl.ds(e0, GE)],
                             sb.at[pl.ds(slot * GE, GE)], sem_i)
            pltpu.async_copy(
                dst_hbm.at[pl.ds(s * (ROWS // NS) + g * GR, GR), :],
                db.at[slot], sem_i)

        def idx_wait():
            pltpu.make_async_copy(src_hbm.at[0, pl.ds(0, GE)],
                                  sb.at[pl.ds(0, GE)], sem_i).wait()
            pltpu.make_async_copy(dst_hbm.at[pl.ds(0, GR), :],
                                  db.at[0], sem_i).wait()

        def gather_issue(slot4, slot2):
            pltpu.async_copy(yp_hbm.at[sb.at[pl.ds(slot4 * GE, GE)]],
                             gb.at[pl.ds(slot2 * GE, GE), :], sem_g)

        def gather_wait():
            pltpu.make_async_copy(yp_hbm.at[sb.at[pl.ds(0, GE)]],
                                  gb.at[pl.ds(0, GE), :], sem_g).wait()

        def scatter_issue(slot4, slot2):
            for j in range(GR):
                pltpu.async_copy(gb.at[pl.ds(slot2 * GE + j * 128, 128), :],
                                 acc.at[db.at[slot4, j]], sem_s, add=True)

        def scatter_wait():
            for j in range(GR):
                pltpu.make_async_copy(gb.at[pl.ds(j * 128, 128), :],
                                      acc.at[db.at[0, 0]], sem_s).wait()

        def per_chunk(k, _):
            chunk = c * 2 + k
            @pl.when(s < NS - 1)
            def _():
                pltpu.sync_copy(zero_hbm,
                                acc.at[pl.ds(s * 3200, 3200), :])
            @pl.when(s == NS - 1)
            def _():
                pltpu.sync_copy(zero_hbm.at[pl.ds(0, 2016), :],
                                acc.at[pl.ds(48000, 2016), :])
            plsc.subcore_barrier()

            idx_load(chunk, 0, 0)
            idx_load(chunk, 1, 1)
            idx_load(chunk, 2, 2)

            def grp(g, _):
                slot3 = lax.rem(g, 3)
                slot6 = lax.rem(g, 6)

                @pl.when(g >= 3)
                def _():
                    scatter_wait()

                @pl.when(g < NG - 3)
                def _():
                    idx_load(chunk, g + 3, lax.rem(g + 3, 6))

                idx_wait()
                gather_issue(slot6, slot3)

                @pl.when(g >= 2)
                def _():
                    gather_wait()
                    scatter_issue(lax.rem(g - 2, 6), lax.rem(g - 2, 3))
                return 0
            lax.fori_loop(0, NG, grp, 0, unroll=False)

            gather_wait()
            scatter_issue(lax.rem(NG - 2, 6), lax.rem(NG - 2, 3))
            gather_wait()
            scatter_issue(lax.rem(NG - 1, 6), lax.rem(NG - 1, 3))
            scatter_wait()
            scatter_wait()
            scatter_wait()

            plsc.subcore_barrier()
            @pl.when(s < NS - 1)
            def _():
                pltpu.sync_copy(acc.at[pl.ds(s * 3200, 3200), :],
                                out_hbm.at[pl.ds(s * 3200, 3200), chunk, :])
            @pl.when(s == NS - 1)
            def _():
                pltpu.sync_copy(acc.at[pl.ds(48000, 2000), :],
                                out_hbm.at[pl.ds(48000, 2000), chunk, :])
            plsc.subcore_barrier()
            return 0
        lax.fori_loop(0, 2, per_chunk, 0)

    return agg_kernel(ypflat, src4f, dstr, zeros)


def _dis_of(degp_ref):
    deg = degp_ref[:, 0] + degp_ref[:, 1] + 1.0
    return lax.rsqrt(deg)


def _mm1(emb, degp, W1):
    def body(emb_ref, degp_ref, w_ref, o_ref):
        dis = _dis_of(degp_ref)
        acc = jnp.dot(emb_ref[...], w_ref[...],
                      preferred_element_type=jnp.float32)
        o_ref[...] = acc * dis[:, None]

    return pl.pallas_call(
        body,
        grid=(NBLK,),
        in_specs=[
            pl.BlockSpec((RB, DIN), lambda i: (i, 0)),
            pl.BlockSpec((RB, NC), lambda i: (i, 0)),
            pl.BlockSpec((DIN, H), lambda i: (0, 0)),
        ],
        out_specs=pl.BlockSpec((RB, H), lambda i: (i, 0)),
        out_shape=jax.ShapeDtypeStruct((N, H), jnp.float32),
    )(emb, degp, W1)


def _mm2(S1, yp, degp, W3, b1r):
    def body(s_ref, yp_ref, degp_ref, w_ref, b_ref, o_ref):
        dis = _dis_of(degp_ref)
        h = jnp.maximum((s_ref[...] + yp_ref[...]) * dis[:, None] + b_ref[...],
                        0.0)
        acc = jnp.dot(h, w_ref[...], preferred_element_type=jnp.float32)
        o_ref[...] = acc * dis[:, None]

    return pl.pallas_call(
        body,
        grid=(NBLK,),
        in_specs=[
            pl.BlockSpec((RB, H), lambda i: (i, 0)),
            pl.BlockSpec((RB, H), lambda i: (i, 0)),
            pl.BlockSpec((RB, NC), lambda i: (i, 0)),
            pl.BlockSpec((H, H), lambda i: (0, 0)),
            pl.BlockSpec((1, H), lambda i: (0, 0)),
        ],
        out_specs=pl.BlockSpec((RB, H), lambda i: (i, 0)),
        out_shape=jax.ShapeDtypeStruct((N, H), jnp.float32),
    )(S1, yp, degp, W3, b1r)


def _pool(S2, zp, degp, b3r, batch2d):
    def body(s_ref, zp_ref, degp_ref, b_ref, bat_ref, o_ref, acc, cnt):
        i = pl.program_id(0)

        @pl.when(i == 0)
        def _():
            acc[...] = jnp.zeros_like(acc)
            cnt[...] = jnp.zeros_like(cnt)

        dis = _dis_of(degp_ref)
        x2 = (s_ref[...] + zp_ref[...]) * dis[:, None] + b_ref[...]
        gi = lax.broadcasted_iota(jnp.int32, (RB, G), 1)
        oh_t = (bat_ref[...] == gi).astype(jnp.float32)
        dn = (((0,), (0,)), ((), ()))
        acc[...] += lax.dot_general(oh_t, x2, dn,
                                    preferred_element_type=jnp.float32)
        cnt[...] += lax.dot_general(oh_t, jnp.ones_like(x2), dn,
                                    preferred_element_type=jnp.float32)

        @pl.when(i == NBLK - 1)
        def _():
            o_ref[...] = acc[...] / jnp.maximum(cnt[...], 1.0)

    return pl.pallas_call(
        body,
        grid=(NBLK,),
        in_specs=[
            pl.BlockSpec((RB, H), lambda i: (i, 0)),
            pl.BlockSpec((RB, H), lambda i: (i, 0)),
            pl.BlockSpec((RB, NC), lambda i: (i, 0)),
            pl.BlockSpec((1, H), lambda i: (0, 0)),
            pl.BlockSpec((RB, 1), lambda i: (i, 0)),
        ],
        out_specs=pl.BlockSpec((G, H), lambda i: (0, 0)),
        out_shape=jax.ShapeDtypeStruct((G, H), jnp.float32),
        scratch_shapes=[
            pltpu.VMEM((G, H), jnp.float32),
            pltpu.VMEM((G, H), jnp.float32),
        ],
    )(S2, zp, degp, b3r, batch2d)


def kernel(emb, edge_index, batch, W1, b1, W3, b3):
    src = edge_index[0].astype(jnp.int32)
    dst = edge_index[1].astype(jnp.int32)
    ar = jnp.arange(PAD, dtype=jnp.int32)
    srcp = jnp.concatenate([src, (ar * 13) % N])
    dstp = jnp.concatenate([dst, N + (ar % 16)])
    src4 = (srcp[None, :] * NCHUNK
            + jnp.arange(NCHUNK, dtype=jnp.int32)[:, None]
            ).reshape(NCHUNK, ROWS, 128)
    dstr = dstp.reshape(ROWS, 128)

    zeros = jnp.zeros((3200, CW), jnp.float32)
    degp = _deg(dstr).T
    yp = _mm1(emb, degp, W1)
    src4f = src4.reshape(NCHUNK, EPAD)
    S1 = _agg(yp.reshape(NCHUNK * N, CW), src4f, dstr, zeros).reshape(N, H)
    zp = _mm2(S1, yp, degp, W3, b1.reshape(1, H))
    S2 = _agg(zp.reshape(NCHUNK * N, CW), src4f, dstr, zeros).reshape(N, H)
    return _pool(S2, zp, degp, b3.reshape(1, H),
                 batch.astype(jnp.int32).reshape(N, 1))


# speedup vs baseline: 1.1392x; 1.0123x over previous
import functools

import jax
import jax.numpy as jnp
from jax import lax
from jax.experimental import pallas as pl
from jax.experimental.pallas import tpu as pltpu
from jax.experimental.pallas import tpu_sc as plsc

N = 50000
E = 800000
DIN = 640
H = 128
G = 64
NC, NS, L = 2, 16, 16
EPAD = 819200
ROWS = EPAD // 128
PAD = EPAD - E
NCHUNK = 4
CW = H // NCHUNK
ACC_ROWS = N + 16
RB = 1000
NBLK = N // RB
DEGN = 51200


def _mesh():
    return plsc.VectorSubcoreMesh(
        core_axis_name="c", subcore_axis_name="s", num_cores=NC,
        num_subcores=NS)


def _deg(dstr):
    @functools.partial(
        pl.kernel,
        out_type=jax.ShapeDtypeStruct((NC, DEGN), jnp.float32),
        mesh=_mesh(),
        scratch_types=[
            pltpu.VMEM_SHARED((DEGN,), jnp.float32),
            pltpu.VMEM((3200,), jnp.float32),
            pltpu.VMEM((128,), jnp.float32),
            pltpu.VMEM((8, 128), jnp.int32),
        ],
        compiler_params=pltpu.CompilerParams(use_tc_tiling_on_sc=False),
    )
    def deg_kernel(dst_hbm, degp_hbm, degacc, zb, ob, db):
        c = lax.axis_index("c")
        s = lax.axis_index("s")

        def fill(j, _):
            zb[pl.ds(j * L, L)] = jnp.zeros((L,), jnp.float32)
            return 0
        lax.fori_loop(0, 3200 // L, fill, 0)

        def fill1(j, _):
            ob[pl.ds(j * L, L)] = jnp.ones((L,), jnp.float32)
            return 0
        lax.fori_loop(0, 128 // L, fill1, 0)

        pltpu.sync_copy(zb, degacc.at[pl.ds(s * 3200, 3200)])
        plsc.subcore_barrier()

        base = c * (ROWS // NC) + s * 200

        def grp(g, _):
            r0 = base + g * 8
            pltpu.sync_copy(dst_hbm.at[pl.ds(r0, 8), :], db)
            for j in range(8):
                pltpu.sync_copy(ob, degacc.at[db.at[j]], add=True)
            return 0
        lax.fori_loop(0, 25, grp, 0)

        plsc.subcore_barrier()
        pltpu.sync_copy(degacc.at[pl.ds(s * 3200, 3200)],
                        degp_hbm.at[c, pl.ds(s * 3200, 3200)])

    return deg_kernel(dstr)


GR = 1
GE = GR * 128
NG = (ROWS // NS) // GR
NSLOT = 6
DG = 4
DS = 5
PF = 3
NIDX = 10


def _agg(ypflat, src4f, dstr, zeros):
    @functools.partial(
        pl.kernel,
        out_type=jax.ShapeDtypeStruct((N, NCHUNK, CW), jnp.float32),
        mesh=_mesh(),
        scratch_types=[
            pltpu.VMEM_SHARED((ACC_ROWS, CW), jnp.float32),
            pltpu.VMEM((NIDX * GE,), jnp.int32),
            pltpu.VMEM((NIDX, GR, 128), jnp.int32),
            pltpu.VMEM((NSLOT * GE, CW), jnp.float32),
            pltpu.SemaphoreType.DMA,
            pltpu.SemaphoreType.DMA,
            pltpu.SemaphoreType.DMA,
        ],
        compiler_params=pltpu.CompilerParams(use_tc_tiling_on_sc=False),
    )
    def agg_kernel(yp_hbm, src_hbm, dst_hbm, zero_hbm, out_hbm,
                   acc, sb, db, gb, sem_i, sem_g, sem_s):
        c = lax.axis_index("c")
        s = lax.axis_index("s")

        def idx_load(chunk, g, slot):
            e0 = (s * (ROWS // NS) + g * GR) * 128
            pltpu.async_copy(src_hbm.at[chunk, pl.ds(e0, GE)],
                             sb.at[pl.ds(slot * GE, GE)], sem_i)
            pltpu.async_copy(
                dst_hbm.at[pl.ds(s * (ROWS // NS) + g * GR, GR), :],
                db.at[slot], sem_i)

        def idx_wait():
            pltpu.make_async_copy(src_hbm.at[0, pl.ds(0, GE)],
                                  sb.at[pl.ds(0, GE)], sem_i).wait()
            pltpu.make_async_copy(dst_hbm.at[pl.ds(0, GR), :],
                                  db.at[0], sem_i).wait()

        def gather_issue(islot, bslot):
            pltpu.async_copy(yp_hbm.at[sb.at[pl.ds(islot * GE, GE)]],
                             gb.at[pl.ds(bslot * GE, GE), :], sem_g)

        def gather_wait():
            pltpu.make_async_copy(yp_hbm.at[sb.at[pl.ds(0, GE)]],
                                  gb.at[pl.ds(0, GE), :], sem_g).wait()

        def scatter_issue(islot, bslot):
            for j in range(GR):
                pltpu.async_copy(gb.at[pl.ds(bslot * GE + j * 128, 128), :],
                                 acc.at[db.at[islot, j]], sem_s, add=True)

        def scatter_wait():
            for j in range(GR):
                pltpu.make_async_copy(gb.at[pl.ds(j * 128, 128), :],
                                      acc.at[db.at[0, 0]], sem_s).wait()

        def per_chunk(k, _):
            chunk = c * 2 + k
            @pl.when(s < NS - 1)
            def _():
                pltpu.sync_copy(zero_hbm,
                                acc.at[pl.ds(s * 3200, 3200), :])
            @pl.when(s == NS - 1)
            def _():
                pltpu.sync_copy(zero_hbm.at[pl.ds(0, 2016), :],
                                acc.at[pl.ds(48000, 2016), :])
            plsc.subcore_barrier()

            for p in range(PF):
                idx_load(chunk, p, p)

            def grp(g, _):
                @pl.when(g >= DS)
                def _():
                    scatter_wait()

                @pl.when(g < NG - PF)
                def _():
                    idx_load(chunk, g + PF, lax.rem(g + PF, NIDX))

                idx_wait()
                gather_issue(lax.rem(g, NIDX), lax.rem(g, NSLOT))

                @pl.when(g >= DG)
                def _():
                    gather_wait()
                    scatter_issue(lax.rem(g - DG, NIDX), lax.rem(g - DG, NSLOT))
                return 0
            lax.fori_loop(0, NG, grp, 0, unroll=False)

            for r in range(DG):
                gather_wait()
                scatter_issue(lax.rem(NG - DG + r, NIDX),
                              lax.rem(NG - DG + r, NSLOT))
            for r in range(DS):
                scatter_wait()

            plsc.subcore_barrier()
            @pl.when(s < NS - 1)
            def _():
                pltpu.sync_copy(acc.at[pl.ds(s * 3200, 3200), :],
                                out_hbm.at[pl.ds(s * 3200, 3200), chunk, :])
            @pl.when(s == NS - 1)
            def _():
                pltpu.sync_copy(acc.at[pl.ds(48000, 2000), :],
                                out_hbm.at[pl.ds(48000, 2000), chunk, :])
            plsc.subcore_barrier()
            return 0
        lax.fori_loop(0, 2, per_chunk, 0)

    return agg_kernel(ypflat, src4f, dstr, zeros)


def _dis_of(degp_ref):
    deg = degp_ref[:, 0] + degp_ref[:, 1] + 1.0
    return lax.rsqrt(deg)


def _mm1(emb, degp, W1):
    def body(emb_ref, degp_ref, w_ref, o_ref):
        dis = _dis_of(degp_ref)
        acc = jnp.dot(emb_ref[...], w_ref[...],
                      preferred_element_type=jnp.float32)
        o_ref[...] = acc * dis[:, None]

    return pl.pallas_call(
        body,
        grid=(NBLK,),
        in_specs=[
            pl.BlockSpec((RB, DIN), lambda i: (i, 0)),
            pl.BlockSpec((RB, NC), lambda i: (i, 0)),
            pl.BlockSpec((DIN, H), lambda i: (0, 0)),
        ],
        out_specs=pl.BlockSpec((RB, H), lambda i: (i, 0)),
        out_shape=jax.ShapeDtypeStruct((N, H), jnp.float32),
    )(emb, degp, W1)


def _mm2(S1, yp, degp, W3, b1r):
    def body(s_ref, yp_ref, degp_ref, w_ref, b_ref, o_ref):
        dis = _dis_of(degp_ref)
        h = jnp.maximum((s_ref[...] + yp_ref[...]) * dis[:, None] + b_ref[...],
                        0.0)
        acc = jnp.dot(h, w_ref[...], preferred_element_type=jnp.float32)
        o_ref[...] = acc * dis[:, None]

    return pl.pallas_call(
        body,
        grid=(NBLK,),
        in_specs=[
            pl.BlockSpec((RB, H), lambda i: (i, 0)),
            pl.BlockSpec((RB, H), lambda i: (i, 0)),
            pl.BlockSpec((RB, NC), lambda i: (i, 0)),
            pl.BlockSpec((H, H), lambda i: (0, 0)),
            pl.BlockSpec((1, H), lambda i: (0, 0)),
        ],
        out_specs=pl.BlockSpec((RB, H), lambda i: (i, 0)),
        out_shape=jax.ShapeDtypeStruct((N, H), jnp.float32),
    )(S1, yp, degp, W3, b1r)


def _pool(S2, zp, degp, b3r, batch2d):
    def body(s_ref, zp_ref, degp_ref, b_ref, bat_ref, o_ref, acc, cnt):
        i = pl.program_id(0)

        @pl.when(i == 0)
        def _():
            acc[...] = jnp.zeros_like(acc)
            cnt[...] = jnp.zeros_like(cnt)

        dis = _dis_of(degp_ref)
        x2 = (s_ref[...] + zp_ref[...]) * dis[:, None] + b_ref[...]
        gi = lax.broadcasted_iota(jnp.int32, (RB, G), 1)
        oh_t = (bat_ref[...] == gi).astype(jnp.float32)
        dn = (((0,), (0,)), ((), ()))
        acc[...] += lax.dot_general(oh_t, x2, dn,
                                    preferred_element_type=jnp.float32)
        cnt[...] += lax.dot_general(oh_t, jnp.ones_like(x2), dn,
                                    preferred_element_type=jnp.float32)

        @pl.when(i == NBLK - 1)
        def _():
            o_ref[...] = acc[...] / jnp.maximum(cnt[...], 1.0)

    return pl.pallas_call(
        body,
        grid=(NBLK,),
        in_specs=[
            pl.BlockSpec((RB, H), lambda i: (i, 0)),
            pl.BlockSpec((RB, H), lambda i: (i, 0)),
            pl.BlockSpec((RB, NC), lambda i: (i, 0)),
            pl.BlockSpec((1, H), lambda i: (0, 0)),
            pl.BlockSpec((RB, 1), lambda i: (i, 0)),
        ],
        out_specs=pl.BlockSpec((G, H), lambda i: (0, 0)),
        out_shape=jax.ShapeDtypeStruct((G, H), jnp.float32),
        scratch_shapes=[
            pltpu.VMEM((G, H), jnp.float32),
            pltpu.VMEM((G, H), jnp.float32),
        ],
    )(S2, zp, degp, b3r, batch2d)


def kernel(emb, edge_index, batch, W1, b1, W3, b3):
    src = edge_index[0].astype(jnp.int32)
    dst = edge_index[1].astype(jnp.int32)
    ar = jnp.arange(PAD, dtype=jnp.int32)
    srcp = jnp.concatenate([src, (ar * 13) % N])
    dstp = jnp.concatenate([dst, N + (ar % 16)])
    src4 = (srcp[None, :] * NCHUNK
            + jnp.arange(NCHUNK, dtype=jnp.int32)[:, None]
            ).reshape(NCHUNK, ROWS, 128)
    dstr = dstp.reshape(ROWS, 128)

    zeros = jnp.zeros((3200, CW), jnp.float32)
    degp = _deg(dstr).T
    yp = _mm1(emb, degp, W1)
    src4f = src4.reshape(NCHUNK, EPAD)
    S1 = _agg(yp.reshape(NCHUNK * N, CW), src4f, dstr, zeros).reshape(N, H)
    zp = _mm2(S1, yp, degp, W3, b1.reshape(1, H))
    S2 = _agg(zp.reshape(NCHUNK * N, CW), src4f, dstr, zeros).reshape(N, H)
    return _pool(S2, zp, degp, b3.reshape(1, H),
                 batch.astype(jnp.int32).reshape(N, 1))
